# baseline (speedup 1.0000x reference)
import functools

import jax
import jax.numpy as jnp
from jax import lax
from jax.experimental import pallas as pl
from jax.experimental.pallas import tpu as pltpu
from jax.experimental.pallas import tpu_sc as plsc

B_, N_, D_, H_, K_ = 4, 2048, 384, 8, 16
HD_ = D_ // H_
CBN = (1.0 + 1e-5) ** -0.5
SCALE = HD_ ** -0.5
TN = 128
TQ = 256
F32 = jnp.float32
_P = jax.lax.Precision.DEFAULT


def _relu(v):
    return jnp.maximum(v, 0.0)


def _dot(a, b, prec=_P):
    return jax.lax.dot_general(a, b, (((1,), (0,)), ((), ())),
                               precision=prec, preferred_element_type=F32)


def _dot_t(a, b, prec=_P):
    return jax.lax.dot_general(a, b, (((1,), (1,)), ((), ())),
                               precision=prec, preferred_element_type=F32)


_NW = 32
_ROWS = B_ * K_ * N_
_PER_W = _ROWS // _NW
_CH = 128
_NCH = _PER_W // _CH


_DC = 512


def _sc_gather_body(xc_hbm, idx_hbm, gx_hbm, idx_v, rows_v, sem1):
    wid = lax.axis_index("s") * 2 + lax.axis_index("c")
    base = wid * _PER_W

    def chunk(j, carry):
        b = base + j * _CH
        pltpu.sync_copy(idx_hbm.at[pl.ds(b, _CH)], idx_v)
        pltpu.async_copy(xc_hbm.at[idx_v], rows_v, sem1).wait()
        pltpu.sync_copy(rows_v, gx_hbm.at[pl.ds(b, _CH)])
        return carry

    lax.fori_loop(0, _NCH, chunk, 0)


def _sc_gather(xc, idxTg):
    mesh = plsc.VectorSubcoreMesh(core_axis_name="c", subcore_axis_name="s")
    f = pl.kernel(
        _sc_gather_body,
        mesh=mesh,
        out_type=jax.ShapeDtypeStruct((_ROWS, _DC), F32),
        scratch_types=[
            pltpu.VMEM((_CH,), jnp.int32),
            pltpu.VMEM((_CH, _DC), F32),
            pltpu.SemaphoreType.DMA,
        ],
    )
    return f(xc, idxTg)


def _local_body(x_ref, gc_ref, xyz4_ref,
                Wm1a_ref, Wm1b_ref, bm1_ref, Wm2_ref, bm2_ref,
                Wpos_ref, bpos_ref, Wl_ref, bl_ref,
                Wq_ref, Wk_ref, Wv_ref,
                xn_ref, q_ref, k_ref, v_ref):
    x = x_ref[0]
    gc = gc_ref[0]
    gx = gc[..., 0:D_].reshape(K_ * TN, D_)

    g4 = gc[..., D_:D_ + 4]
    rel = g4 - g4[0:1]
    d2 = jnp.sum(rel * rel, -1, keepdims=True)
    lane4 = jax.lax.broadcasted_iota(jnp.int32, (K_, TN, 4), 2)
    rel4 = jnp.where(lane4 == 3, d2, rel).reshape(K_ * TN, 4)

    pos = _dot(rel4, Wpos_ref[...]) + bpos_ref[...]
    keyf = gx + pos

    a1 = _dot(_relu(keyf * CBN), Wm1b_ref[...])
    tq = _dot(_relu(x * CBN), Wm1a_ref[...])
    h1 = (a1.reshape(K_, TN, D_) + tq[None] + bm1_ref[...]).reshape(K_ * TN, D_)
    logits = (_dot(_relu(h1 * CBN), Wm2_ref[...]) + bm2_ref[...]) * SCALE

    lane = jax.lax.broadcasted_iota(jnp.int32, (H_, D_), 1)
    hid = jax.lax.broadcasted_iota(jnp.int32, (H_, D_), 0)
    E = (lane // HD_ == hid).astype(F32)

    m = logits[0:TN]
    for kk in range(1, K_):
        m = jnp.maximum(m, logits[kk * TN:(kk + 1) * TN])
    s = jnp.zeros((TN, H_), F32)
    acc = jnp.zeros((TN, D_), F32)
    for kk in range(K_):
        p = jnp.exp(logits[kk * TN:(kk + 1) * TN] - m)
        s = s + p
        acc = acc + _dot(p, E) * keyf[kk * TN:(kk + 1) * TN]
    out = acc / _dot(s, E)

    o = _dot(_relu(out * CBN), Wl_ref[...]) + bl_ref[...]
    xn = x + o
    xn_ref[0] = xn
    q_ref[0] = _dot(xn, Wq_ref[...]) * SCALE
    k_ref[0] = _dot(xn, Wk_ref[...])
    v_ref[0] = _dot(xn, Wv_ref[...])


def _global_body(xn_ref, q_ref, k_ref, v_ref, Wg_ref, bg_ref, out_ref):
    q = q_ref[0]
    kf = k_ref[0]
    vf = v_ref[0]
    acc = jnp.zeros((TQ, D_), F32)
    for h in range(H_):
        sl = slice(h * HD_, (h + 1) * HD_)
        sc = _dot_t(q[:, sl], kf[:, sl])
        m = jnp.max(sc, axis=1, keepdims=True)
        p = jnp.exp(sc - m)
        den = jnp.sum(p, axis=1, keepdims=True)
        sv = _dot(p, vf[:, sl])
        acc = acc + _dot(sv / den, Wg_ref[sl, :])
    out_ref[0] = xn_ref[0] + acc + bg_ref[...]


def _local_call(x, gcT, xyz4, Wm1a, Wm1b, bm1, Wm2, bm2,
                Wpos, bpos, Wl, bl, Wq, Wk, Wv):
    grid = (B_, N_ // TN)
    full = lambda shape: pl.BlockSpec(shape, lambda b, n: (0,) * len(shape))
    out_bs = pl.BlockSpec((1, TN, D_), lambda b, n: (b, n, 0))
    return pl.pallas_call(
        _local_body,
        grid=grid,
        in_specs=[
            pl.BlockSpec((1, TN, D_), lambda b, n: (b, n, 0)),
            pl.BlockSpec((1, K_, TN, _DC), lambda b, n: (b, 0, n, 0)),
            pl.BlockSpec((1, TN, 4), lambda b, n: (b, n, 0)),
            full((D_, D_)), full((D_, D_)), full((1, D_)),
            full((D_, H_)), full((1, H_)),
            full((4, D_)), full((1, D_)),
            full((D_, D_)), full((1, D_)),
            full((D_, D_)), full((D_, D_)), full((D_, D_)),
        ],
        out_specs=[out_bs, out_bs, out_bs, out_bs],
        out_shape=[jax.ShapeDtypeStruct((B_, N_, D_), F32)] * 4,
    )(x, gcT, xyz4, Wm1a, Wm1b, bm1, Wm2, bm2, Wpos, bpos, Wl, bl, Wq, Wk, Wv)


def _global_call(xn, q, k, v, Wg, bg):
    grid = (B_, N_ // TQ)
    tile = pl.BlockSpec((1, TQ, D_), lambda b, n: (b, n, 0))
    row = pl.BlockSpec((1, N_, D_), lambda b, n: (b, 0, 0))
    return pl.pallas_call(
        _global_body,
        grid=grid,
        in_specs=[tile, tile, row, row,
                  pl.BlockSpec((D_, D_), lambda b, n: (0, 0)),
                  pl.BlockSpec((1, D_), lambda b, n: (0, 0))],
        out_specs=tile,
        out_shape=jax.ShapeDtypeStruct((B_, N_, D_), F32),
    )(xn, q, k, v, Wg, bg)


def kernel(x, xyz, W_pos, b_pos, W_m1, b_m1, W_m2, b_m2,
           W_lproj, b_lproj, W_q, W_k, W_v, W_gproj, b_gproj):
    sq = (-2.0 * jnp.einsum('bnd,bmd->bnm', xyz, xyz)
          + jnp.sum(xyz ** 2, -1)[:, :, None]
          + jnp.sum(xyz ** 2, -1)[:, None, :])
    _, idx = jax.lax.top_k(-sq, K_)

    idxT = jnp.swapaxes(idx, 1, 2)
    idxTg = (idxT + (jnp.arange(B_, dtype=idx.dtype) * N_)[:, None, None])
    xyz4 = jnp.pad(xyz, ((0, 0), (0, 0), (0, 1)))
    xc = jnp.concatenate(
        [x, jnp.pad(xyz, ((0, 0), (0, 0), (0, _DC - D_ - 3)))], axis=-1)
    gc_flat = _sc_gather(xc.reshape(B_ * N_, _DC),
                         idxTg.reshape(_ROWS).astype(jnp.int32))
    gcT = gc_flat.reshape(B_, K_, N_, _DC)

    r2 = lambda a: a.reshape(1, -1)
    xn, q, k, v = _local_call(
        x, gcT, xyz4,
        W_m1[:D_], W_m1[D_:], r2(b_m1), W_m2, r2(b_m2),
        W_pos, r2(b_pos), W_lproj, r2(b_lproj), W_q, W_k, W_v)

    return _global_call(xn, q, k, v, W_gproj, r2(b_gproj))


# speedup vs baseline: 3.4452x; 3.4452x over previous
import functools

import jax
import jax.numpy as jnp
from jax import lax
from jax.experimental import pallas as pl
from jax.experimental.pallas import tpu as pltpu
from jax.experimental.pallas import tpu_sc as plsc

B_, N_, D_, H_, K_ = 4, 2048, 384, 8, 16
HD_ = D_ // H_
CBN = (1.0 + 1e-5) ** -0.5
SCALE = HD_ ** -0.5
TN = 128
TQ = 256
F32 = jnp.float32
_P = jax.lax.Precision.DEFAULT


def _relu(v):
    return jnp.maximum(v, 0.0)


def _dot(a, b, prec=_P):
    return jax.lax.dot_general(a, b, (((1,), (0,)), ((), ())),
                               precision=prec, preferred_element_type=F32)


def _dot_t(a, b, prec=_P):
    return jax.lax.dot_general(a, b, (((1,), (1,)), ((), ())),
                               precision=prec, preferred_element_type=F32)


_NW = 32
_ROWS = B_ * K_ * N_
_PER_W = _ROWS // _NW
_CH = 128
_NCH = _PER_W // _CH


_DC = 512


def _sc_gather_body(xc_hbm, idx_hbm, gx_hbm, idx_v, rows_v, sem1):
    wid = lax.axis_index("s") * 2 + lax.axis_index("c")
    base = wid * _PER_W

    def chunk(j, carry):
        b = base + j * _CH
        pltpu.sync_copy(idx_hbm.at[pl.ds(b, _CH)], idx_v)
        pltpu.async_copy(xc_hbm.at[idx_v], rows_v, sem1).wait()
        pltpu.sync_copy(rows_v, gx_hbm.at[pl.ds(b, _CH)])
        return carry

    lax.fori_loop(0, _NCH, chunk, 0)


def _sc_gather(xc, idxTg):
    mesh = plsc.VectorSubcoreMesh(core_axis_name="c", subcore_axis_name="s")
    f = pl.kernel(
        _sc_gather_body,
        mesh=mesh,
        out_type=jax.ShapeDtypeStruct((_ROWS, _DC), F32),
        scratch_types=[
            pltpu.VMEM((_CH,), jnp.int32),
            pltpu.VMEM((_CH, _DC), F32),
            pltpu.SemaphoreType.DMA,
        ],
    )
    return f(xc, idxTg)


TR = 256


def _topk_body(xyz4_ref, xyzall_ref, n2_ref, idx_ref):
    b = pl.program_id(0)
    xt = xyz4_ref[0]
    n2t = jnp.sum(xt * xt, axis=1, keepdims=True)
    d = n2t + n2_ref[0] - 2.0 * _dot_t(xt, xyzall_ref[0])
    lane_n = jax.lax.broadcasted_iota(jnp.int32, (TR, N_), 1)
    lane_k = jax.lax.broadcasted_iota(jnp.int32, (TR, K_), 1)
    idxs = jnp.zeros((TR, K_), jnp.int32)
    for kk in range(K_):
        m = jnp.min(d, axis=1, keepdims=True)
        cand = jnp.where(d == m, lane_n, N_)
        a = jnp.min(cand, axis=1, keepdims=True)
        idxs = jnp.where(lane_k == kk, a + b * N_, idxs)
        d = jnp.where(lane_n == a, float('inf'), d)
    idx_ref[0] = idxs


def _topk_call(xyz4, n2):
    grid = (B_, N_ // TR)
    return pl.pallas_call(
        _topk_body,
        grid=grid,
        in_specs=[
            pl.BlockSpec((1, TR, 4), lambda b, n: (b, n, 0)),
            pl.BlockSpec((1, N_, 4), lambda b, n: (b, 0, 0)),
            pl.BlockSpec((1, 1, N_), lambda b, n: (b, 0, 0)),
        ],
        out_specs=pl.BlockSpec((1, TR, K_), lambda b, n: (b, n, 0)),
        out_shape=jax.ShapeDtypeStruct((B_, N_, K_), jnp.int32),
    )(xyz4, xyz4, n2)


def _local_body(x_ref, gc_ref, xyz4_ref,
                Wm1a_ref, Wm1b_ref, bm1_ref, Wm2_ref, bm2_ref,
                Wpos_ref, bpos_ref, Wl_ref, bl_ref,
                Wq_ref, Wk_ref, Wv_ref,
                xn_ref, q_ref, k_ref, v_ref):
    x = x_ref[0]
    gc = gc_ref[0]
    gx = gc[..., 0:D_].reshape(K_ * TN, D_)

    g4 = gc[..., D_:D_ + 4]
    rel = g4 - g4[0:1]
    d2 = jnp.sum(rel * rel, -1, keepdims=True)
    lane4 = jax.lax.broadcasted_iota(jnp.int32, (K_, TN, 4), 2)
    rel4 = jnp.where(lane4 == 3, d2, rel).reshape(K_ * TN, 4)

    pos = _dot(rel4, Wpos_ref[...]) + bpos_ref[...]
    keyf = gx + pos

    a1 = _dot(_relu(keyf * CBN), Wm1b_ref[...])
    tq = _dot(_relu(x * CBN), Wm1a_ref[...])
    h1 = (a1.reshape(K_, TN, D_) + tq[None] + bm1_ref[...]).reshape(K_ * TN, D_)
    logits = (_dot(_relu(h1 * CBN), Wm2_ref[...]) + bm2_ref[...]) * SCALE

    lane = jax.lax.broadcasted_iota(jnp.int32, (H_, D_), 1)
    hid = jax.lax.broadcasted_iota(jnp.int32, (H_, D_), 0)
    E = (lane // HD_ == hid).astype(F32)

    m = logits[0:TN]
    for kk in range(1, K_):
        m = jnp.maximum(m, logits[kk * TN:(kk + 1) * TN])
    s = jnp.zeros((TN, H_), F32)
    acc = jnp.zeros((TN, D_), F32)
    for kk in range(K_):
        p = jnp.exp(logits[kk * TN:(kk + 1) * TN] - m)
        s = s + p
        acc = acc + _dot(p, E) * keyf[kk * TN:(kk + 1) * TN]
    out = acc / _dot(s, E)

    o = _dot(_relu(out * CBN), Wl_ref[...]) + bl_ref[...]
    xn = x + o
    xn_ref[0] = xn
    q_ref[0] = _dot(xn, Wq_ref[...]) * SCALE
    k_ref[0] = _dot(xn, Wk_ref[...])
    v_ref[0] = _dot(xn, Wv_ref[...])


def _global_body(xn_ref, q_ref, k_ref, v_ref, Wg_ref, bg_ref, out_ref):
    q = q_ref[0]
    kf = k_ref[0]
    vf = v_ref[0]
    acc = jnp.zeros((TQ, D_), F32)
    for h in range(H_):
        sl = slice(h * HD_, (h + 1) * HD_)
        sc = _dot_t(q[:, sl], kf[:, sl])
        m = jnp.max(sc, axis=1, keepdims=True)
        p = jnp.exp(sc - m)
        den = jnp.sum(p, axis=1, keepdims=True)
        sv = _dot(p, vf[:, sl])
        acc = acc + _dot(sv / den, Wg_ref[sl, :])
    out_ref[0] = xn_ref[0] + acc + bg_ref[...]


def _local_call(x, gcT, xyz4, Wm1a, Wm1b, bm1, Wm2, bm2,
                Wpos, bpos, Wl, bl, Wq, Wk, Wv):
    grid = (B_, N_ // TN)
    full = lambda shape: pl.BlockSpec(shape, lambda b, n: (0,) * len(shape))
    out_bs = pl.BlockSpec((1, TN, D_), lambda b, n: (b, n, 0))
    return pl.pallas_call(
        _local_body,
        grid=grid,
        in_specs=[
            pl.BlockSpec((1, TN, D_), lambda b, n: (b, n, 0)),
            pl.BlockSpec((1, K_, TN, _DC), lambda b, n: (b, 0, n, 0)),
            pl.BlockSpec((1, TN, 4), lambda b, n: (b, n, 0)),
            full((D_, D_)), full((D_, D_)), full((1, D_)),
            full((D_, H_)), full((1, H_)),
            full((4, D_)), full((1, D_)),
            full((D_, D_)), full((1, D_)),
            full((D_, D_)), full((D_, D_)), full((D_, D_)),
        ],
        out_specs=[out_bs, out_bs, out_bs, out_bs],
        out_shape=[jax.ShapeDtypeStruct((B_, N_, D_), F32)] * 4,
    )(x, gcT, xyz4, Wm1a, Wm1b, bm1, Wm2, bm2, Wpos, bpos, Wl, bl, Wq, Wk, Wv)


def _global_call(xn, q, k, v, Wg, bg):
    grid = (B_, N_ // TQ)
    tile = pl.BlockSpec((1, TQ, D_), lambda b, n: (b, n, 0))
    row = pl.BlockSpec((1, N_, D_), lambda b, n: (b, 0, 0))
    return pl.pallas_call(
        _global_body,
        grid=grid,
        in_specs=[tile, tile, row, row,
                  pl.BlockSpec((D_, D_), lambda b, n: (0, 0)),
                  pl.BlockSpec((1, D_), lambda b, n: (0, 0))],
        out_specs=tile,
        out_shape=jax.ShapeDtypeStruct((B_, N_, D_), F32),
    )(xn, q, k, v, Wg, bg)


def kernel(x, xyz, W_pos, b_pos, W_m1, b_m1, W_m2, b_m2,
           W_lproj, b_lproj, W_q, W_k, W_v, W_gproj, b_gproj):
    xyz4 = jnp.pad(xyz, ((0, 0), (0, 0), (0, 1)))
    n2 = jnp.sum(xyz * xyz, -1)[:, None, :]
    idxg = _topk_call(xyz4, n2)
    idxTg = jnp.swapaxes(idxg, 1, 2)
    xc = jnp.concatenate(
        [x, jnp.pad(xyz, ((0, 0), (0, 0), (0, _DC - D_ - 3)))], axis=-1)
    gc_flat = _sc_gather(xc.reshape(B_ * N_, _DC),
                         idxTg.reshape(_ROWS).astype(jnp.int32))
    gcT = gc_flat.reshape(B_, K_, N_, _DC)

    r2 = lambda a: a.reshape(1, -1)
    xn, q, k, v = _local_call(
        x, gcT, xyz4,
        W_m1[:D_], W_m1[D_:], r2(b_m1), W_m2, r2(b_m2),
        W_pos, r2(b_pos), W_lproj, r2(b_lproj), W_q, W_k, W_v)

    return _global_call(xn, q, k, v, W_gproj, r2(b_gproj))


# speedup vs baseline: 3.5323x; 1.0253x over previous
import functools

import jax
import jax.numpy as jnp
from jax import lax
from jax.experimental import pallas as pl
from jax.experimental.pallas import tpu as pltpu
from jax.experimental.pallas import tpu_sc as plsc

B_, N_, D_, H_, K_ = 4, 2048, 384, 8, 16
HD_ = D_ // H_
CBN = (1.0 + 1e-5) ** -0.5
SCALE = HD_ ** -0.5
TN = 256
TQ = 256
F32 = jnp.float32
_P = jax.lax.Precision.DEFAULT


def _relu(v):
    return jnp.maximum(v, 0.0)


def _dot(a, b, prec=_P):
    return jax.lax.dot_general(a, b, (((1,), (0,)), ((), ())),
                               precision=prec, preferred_element_type=F32)


def _dot_t(a, b, prec=_P):
    return jax.lax.dot_general(a, b, (((1,), (1,)), ((), ())),
                               precision=prec, preferred_element_type=F32)


_NW = 32
_ROWS = B_ * K_ * N_
_PER_W = _ROWS // _NW
_CH = 128
_NCH = _PER_W // _CH


_DC = 512


def _sc_gather_body(xc_hbm, idx_hbm, gx_hbm, idx_v, rows_v, sem1):
    wid = lax.axis_index("s") * 2 + lax.axis_index("c")
    base = wid * _PER_W

    def chunk(j, carry):
        b = base + j * _CH
        pltpu.sync_copy(idx_hbm.at[pl.ds(b, _CH)], idx_v)
        pltpu.async_copy(xc_hbm.at[idx_v], rows_v, sem1).wait()
        pltpu.sync_copy(rows_v, gx_hbm.at[pl.ds(b, _CH)])
        return carry

    lax.fori_loop(0, _NCH, chunk, 0)


def _sc_gather(xc, idxTg):
    mesh = plsc.VectorSubcoreMesh(core_axis_name="c", subcore_axis_name="s")
    f = pl.kernel(
        _sc_gather_body,
        mesh=mesh,
        out_type=jax.ShapeDtypeStruct((_ROWS, _DC), F32),
        scratch_types=[
            pltpu.VMEM((_CH,), jnp.int32),
            pltpu.VMEM((_CH, _DC), F32),
            pltpu.SemaphoreType.DMA,
        ],
    )
    return f(xc, idxTg)


TR = 256


def _topk_body(xyz4_ref, xyzall_ref, n2_ref, idx_ref):
    b = pl.program_id(0)
    xt = xyz4_ref[0]
    n2t = jnp.sum(xt * xt, axis=1, keepdims=True)
    d = n2t + n2_ref[0] - 2.0 * _dot_t(xt, xyzall_ref[0])
    lane_n = jax.lax.broadcasted_iota(jnp.int32, (TR, N_), 1)
    lane_k = jax.lax.broadcasted_iota(jnp.int32, (TR, K_), 1)
    idxs = jnp.zeros((TR, K_), jnp.int32)
    for kk in range(K_):
        m = jnp.min(d, axis=1, keepdims=True)
        cand = jnp.where(d == m, lane_n, N_)
        a = jnp.min(cand, axis=1, keepdims=True)
        idxs = jnp.where(lane_k == kk, a + b * N_, idxs)
        d = jnp.where(lane_n == a, float('inf'), d)
    idx_ref[0] = idxs


def _topk_call(xyz4, n2):
    grid = (B_, N_ // TR)
    return pl.pallas_call(
        _topk_body,
        grid=grid,
        in_specs=[
            pl.BlockSpec((1, TR, 4), lambda b, n: (b, n, 0)),
            pl.BlockSpec((1, N_, 4), lambda b, n: (b, 0, 0)),
            pl.BlockSpec((1, 1, N_), lambda b, n: (b, 0, 0)),
        ],
        out_specs=pl.BlockSpec((1, TR, K_), lambda b, n: (b, n, 0)),
        out_shape=jax.ShapeDtypeStruct((B_, N_, K_), jnp.int32),
    )(xyz4, xyz4, n2)


def _local_body(x_ref, gc_ref, xyz4_ref,
                Wm1a_ref, Wm1b_ref, bm1_ref, Wm2_ref, bm2_ref,
                Wpos_ref, bpos_ref, Wl_ref, bl_ref,
                Wq_ref, Wk_ref, Wv_ref,
                xn_ref, q_ref, k_ref, v_ref):
    x = x_ref[0]
    gc = gc_ref[0]
    gx = gc[..., 0:D_].reshape(K_ * TN, D_)

    g4 = gc[..., D_:D_ + 4]
    rel = g4 - g4[0:1]
    d2 = jnp.sum(rel * rel, -1, keepdims=True)
    lane4 = jax.lax.broadcasted_iota(jnp.int32, (K_, TN, 4), 2)
    rel4 = jnp.where(lane4 == 3, d2, rel).reshape(K_ * TN, 4)

    pos = _dot(rel4, Wpos_ref[...]) + bpos_ref[...]
    keyf = gx + pos

    a1 = _dot(_relu(keyf * CBN), Wm1b_ref[...])
    tq = _dot(_relu(x * CBN), Wm1a_ref[...])
    h1 = (a1.reshape(K_, TN, D_) + tq[None] + bm1_ref[...]).reshape(K_ * TN, D_)
    logits = (_dot(_relu(h1 * CBN), Wm2_ref[...]) + bm2_ref[...]) * SCALE

    lane = jax.lax.broadcasted_iota(jnp.int32, (H_, D_), 1)
    hid = jax.lax.broadcasted_iota(jnp.int32, (H_, D_), 0)
    E = (lane // HD_ == hid).astype(F32)

    m = logits[0:TN]
    for kk in range(1, K_):
        m = jnp.maximum(m, logits[kk * TN:(kk + 1) * TN])
    s = jnp.zeros((TN, H_), F32)
    acc = jnp.zeros((TN, D_), F32)
    for kk in range(K_):
        p = jnp.exp(logits[kk * TN:(kk + 1) * TN] - m)
        s = s + p
        acc = acc + _dot(p, E) * keyf[kk * TN:(kk + 1) * TN]
    out = acc / _dot(s, E)

    o = _dot(_relu(out * CBN), Wl_ref[...]) + bl_ref[...]
    xn = x + o
    xn_ref[0] = xn
    q_ref[0] = _dot(xn, Wq_ref[...]) * SCALE
    k_ref[0] = _dot(xn, Wk_ref[...])
    v_ref[0] = _dot(xn, Wv_ref[...])


def _global_body(xn_ref, q_ref, k_ref, v_ref, Wg_ref, bg_ref, out_ref):
    q = q_ref[0]
    kf = k_ref[0]
    vf = v_ref[0]
    acc = jnp.zeros((TQ, D_), F32)
    for h in range(H_):
        sl = slice(h * HD_, (h + 1) * HD_)
        sc = _dot_t(q[:, sl], kf[:, sl])
        m = jnp.max(sc, axis=1, keepdims=True)
        p = jnp.exp(sc - m)
        den = jnp.sum(p, axis=1, keepdims=True)
        sv = _dot(p, vf[:, sl])
        acc = acc + _dot(sv / den, Wg_ref[sl, :])
    out_ref[0] = xn_ref[0] + acc + bg_ref[...]


def _local_call(x, gcT, xyz4, Wm1a, Wm1b, bm1, Wm2, bm2,
                Wpos, bpos, Wl, bl, Wq, Wk, Wv):
    grid = (B_, N_ // TN)
    full = lambda shape: pl.BlockSpec(shape, lambda b, n: (0,) * len(shape))
    out_bs = pl.BlockSpec((1, TN, D_), lambda b, n: (b, n, 0))
    return pl.pallas_call(
        _local_body,
        grid=grid,
        in_specs=[
            pl.BlockSpec((1, TN, D_), lambda b, n: (b, n, 0)),
            pl.BlockSpec((1, K_, TN, _DC), lambda b, n: (b, 0, n, 0)),
            pl.BlockSpec((1, TN, 4), lambda b, n: (b, n, 0)),
            full((D_, D_)), full((D_, D_)), full((1, D_)),
            full((D_, H_)), full((1, H_)),
            full((4, D_)), full((1, D_)),
            full((D_, D_)), full((1, D_)),
            full((D_, D_)), full((D_, D_)), full((D_, D_)),
        ],
        out_specs=[out_bs, out_bs, out_bs, out_bs],
        out_shape=[jax.ShapeDtypeStruct((B_, N_, D_), F32)] * 4,
    )(x, gcT, xyz4, Wm1a, Wm1b, bm1, Wm2, bm2, Wpos, bpos, Wl, bl, Wq, Wk, Wv)


def _global_call(xn, q, k, v, Wg, bg):
    grid = (B_, N_ // TQ)
    tile = pl.BlockSpec((1, TQ, D_), lambda b, n: (b, n, 0))
    row = pl.BlockSpec((1, N_, D_), lambda b, n: (b, 0, 0))
    return pl.pallas_call(
        _global_body,
        grid=grid,
        in_specs=[tile, tile, row, row,
                  pl.BlockSpec((D_, D_), lambda b, n: (0, 0)),
                  pl.BlockSpec((1, D_), lambda b, n: (0, 0))],
        out_specs=tile,
        out_shape=jax.ShapeDtypeStruct((B_, N_, D_), F32),
    )(xn, q, k, v, Wg, bg)


def kernel(x, xyz, W_pos, b_pos, W_m1, b_m1, W_m2, b_m2,
           W_lproj, b_lproj, W_q, W_k, W_v, W_gproj, b_gproj):
    xyz4 = jnp.pad(xyz, ((0, 0), (0, 0), (0, 1)))
    n2 = jnp.sum(xyz * xyz, -1)[:, None, :]
    idxg = _topk_call(xyz4, n2)
    idxTg = jnp.swapaxes(idxg, 1, 2)
    xc = jnp.concatenate(
        [x, jnp.pad(xyz, ((0, 0), (0, 0), (0, _DC - D_ - 3)))], axis=-1)
    gc_flat = _sc_gather(xc.reshape(B_ * N_, _DC),
                         idxTg.reshape(_ROWS).astype(jnp.int32))
    gcT = gc_flat.reshape(B_, K_, N_, _DC)

    r2 = lambda a: a.reshape(1, -1)
    xn, q, k, v = _local_call(
        x, gcT, xyz4,
        W_m1[:D_], W_m1[D_:], r2(b_m1), W_m2, r2(b_m2),
        W_pos, r2(b_pos), W_lproj, r2(b_lproj), W_q, W_k, W_v)

    return _global_call(xn, q, k, v, W_gproj, r2(b_gproj))


# speedup vs baseline: 3.5564x; 1.0068x over previous
import functools

import jax
import jax.numpy as jnp
from jax import lax
from jax.experimental import pallas as pl
from jax.experimental.pallas import tpu as pltpu
from jax.experimental.pallas import tpu_sc as plsc

B_, N_, D_, H_, K_ = 4, 2048, 384, 8, 16
HD_ = D_ // H_
CBN = (1.0 + 1e-5) ** -0.5
SCALE = HD_ ** -0.5
TN = 256
TQ = 256
F32 = jnp.float32
_P = jax.lax.Precision.DEFAULT


def _relu(v):
    return jnp.maximum(v, 0.0)


def _dot(a, b, prec=_P):
    return jax.lax.dot_general(a, b, (((1,), (0,)), ((), ())),
                               precision=prec, preferred_element_type=F32)


def _dot_t(a, b, prec=_P):
    return jax.lax.dot_general(a, b, (((1,), (1,)), ((), ())),
                               precision=prec, preferred_element_type=F32)


_NW = 32
_ROWS = B_ * K_ * N_
_PER_W = _ROWS // _NW
_CH = 128
_NCH = _PER_W // _CH


_DC = 512


def _sc_gather_body(xc_hbm, idx_hbm, gx_hbm, idx_v, rows_v, sem1):
    wid = lax.axis_index("s") * 2 + lax.axis_index("c")
    base = wid * _PER_W

    def chunk(j, carry):
        b = base + j * _CH
        pltpu.sync_copy(idx_hbm.at[pl.ds(b, _CH)], idx_v)
        pltpu.async_copy(xc_hbm.at[idx_v], rows_v, sem1).wait()
        pltpu.sync_copy(rows_v, gx_hbm.at[pl.ds(b, _CH)])
        return carry

    lax.fori_loop(0, _NCH, chunk, 0)


def _sc_gather(xc, idxTg):
    mesh = plsc.VectorSubcoreMesh(core_axis_name="c", subcore_axis_name="s")
    f = pl.kernel(
        _sc_gather_body,
        mesh=mesh,
        out_type=jax.ShapeDtypeStruct((_ROWS, _DC), F32),
        scratch_types=[
            pltpu.VMEM((_CH,), jnp.int32),
            pltpu.VMEM((_CH, _DC), F32),
            pltpu.SemaphoreType.DMA,
        ],
    )
    return f(xc, idxTg)


TR = 256


def _topk_body(xyz4_ref, xyzall_ref, n2_ref, idx_ref):
    b = pl.program_id(0)
    xt = xyz4_ref[0]
    n2t = jnp.sum(xt * xt, axis=1, keepdims=True)
    d = n2t + n2_ref[0] - 2.0 * _dot_t(xt, xyzall_ref[0])
    lane_n = jax.lax.broadcasted_iota(jnp.int32, (TR, N_), 1)
    lane_k = jax.lax.broadcasted_iota(jnp.int32, (TR, K_), 1)
    idxs = jnp.zeros((TR, K_), jnp.int32)
    for kk in range(K_):
        m = jnp.min(d, axis=1, keepdims=True)
        cand = jnp.where(d == m, lane_n, N_)
        a = jnp.min(cand, axis=1, keepdims=True)
        idxs = jnp.where(lane_k == kk, a + b * N_, idxs)
        d = jnp.where(lane_n == a, float('inf'), d)
    idx_ref[0] = idxs


def _topk_call(xyz4, n2):
    grid = (B_, N_ // TR)
    return pl.pallas_call(
        _topk_body,
        grid=grid,
        in_specs=[
            pl.BlockSpec((1, TR, 4), lambda b, n: (b, n, 0)),
            pl.BlockSpec((1, N_, 4), lambda b, n: (b, 0, 0)),
            pl.BlockSpec((1, 1, N_), lambda b, n: (b, 0, 0)),
        ],
        out_specs=pl.BlockSpec((1, TR, K_), lambda b, n: (b, n, 0)),
        out_shape=jax.ShapeDtypeStruct((B_, N_, K_), jnp.int32),
    )(xyz4, xyz4, n2)


def _local_body(x_ref, gc_ref, xyz4_ref,
                Wm1a_ref, Wm1b_ref, bm1_ref, Wm2_ref, bm2_ref,
                Wpos_ref, bpos_ref, Wl_ref, bl_ref,
                Wq_ref, Wk_ref, Wv_ref,
                xn_ref, q_ref, k_ref, v_ref):
    x = x_ref[0]
    gc = gc_ref[0]
    gx = gc[..., 0:D_].reshape(K_ * TN, D_)

    g4 = gc[..., D_:D_ + 4]
    rel = g4 - g4[0:1]
    d2 = jnp.sum(rel * rel, -1, keepdims=True)
    lane4 = jax.lax.broadcasted_iota(jnp.int32, (K_, TN, 4), 2)
    rel4 = jnp.where(lane4 == 3, d2, rel).reshape(K_ * TN, 4)

    pos = _dot(rel4, Wpos_ref[...]) + bpos_ref[...]
    keyf = gx + pos

    a1 = _dot(_relu(keyf * CBN).astype(jnp.bfloat16),
              Wm1b_ref[...].astype(jnp.bfloat16))
    tq = _dot(_relu(x * CBN), Wm1a_ref[...])
    h1 = (a1.reshape(K_, TN, D_) + tq[None] + bm1_ref[...]).reshape(K_ * TN, D_)
    logits = (_dot(_relu(h1 * CBN), Wm2_ref[...]) + bm2_ref[...]) * SCALE

    lane = jax.lax.broadcasted_iota(jnp.int32, (H_, D_), 1)
    hid = jax.lax.broadcasted_iota(jnp.int32, (H_, D_), 0)
    E = (lane // HD_ == hid).astype(F32)

    m = logits[0:TN]
    for kk in range(1, K_):
        m = jnp.maximum(m, logits[kk * TN:(kk + 1) * TN])
    s = jnp.zeros((TN, H_), F32)
    acc = jnp.zeros((TN, D_), F32)
    for kk in range(K_):
        p = jnp.exp(logits[kk * TN:(kk + 1) * TN] - m)
        s = s + p
        acc = acc + _dot(p, E) * keyf[kk * TN:(kk + 1) * TN]
    out = acc / _dot(s, E)

    o = _dot(_relu(out * CBN), Wl_ref[...]) + bl_ref[...]
    xn = x + o
    xn_ref[0] = xn
    q_ref[0] = _dot(xn, Wq_ref[...]) * SCALE
    k_ref[0] = _dot(xn, Wk_ref[...])
    v_ref[0] = _dot(xn, Wv_ref[...])


def _global_body(xn_ref, q_ref, k_ref, v_ref, Wg_ref, bg_ref, out_ref):
    q = q_ref[0]
    kf = k_ref[0]
    vf = v_ref[0]
    acc = jnp.zeros((TQ, D_), F32)
    for h in range(H_):
        sl = slice(h * HD_, (h + 1) * HD_)
        sc = _dot_t(q[:, sl], kf[:, sl])
        m = jnp.max(sc, axis=1, keepdims=True)
        p = jnp.exp(sc - m)
        den = jnp.sum(p, axis=1, keepdims=True)
        sv = _dot(p, vf[:, sl])
        acc = acc + _dot(sv / den, Wg_ref[sl, :])
    out_ref[0] = xn_ref[0] + acc + bg_ref[...]


def _local_call(x, gcT, xyz4, Wm1a, Wm1b, bm1, Wm2, bm2,
                Wpos, bpos, Wl, bl, Wq, Wk, Wv):
    grid = (B_, N_ // TN)
    full = lambda shape: pl.BlockSpec(shape, lambda b, n: (0,) * len(shape))
    out_bs = pl.BlockSpec((1, TN, D_), lambda b, n: (b, n, 0))
    return pl.pallas_call(
        _local_body,
        grid=grid,
        in_specs=[
            pl.BlockSpec((1, TN, D_), lambda b, n: (b, n, 0)),
            pl.BlockSpec((1, K_, TN, _DC), lambda b, n: (b, 0, n, 0)),
            pl.BlockSpec((1, TN, 4), lambda b, n: (b, n, 0)),
            full((D_, D_)), full((D_, D_)), full((1, D_)),
            full((D_, H_)), full((1, H_)),
            full((4, D_)), full((1, D_)),
            full((D_, D_)), full((1, D_)),
            full((D_, D_)), full((D_, D_)), full((D_, D_)),
        ],
        out_specs=[out_bs, out_bs, out_bs, out_bs],
        out_shape=[jax.ShapeDtypeStruct((B_, N_, D_), F32)] * 4,
    )(x, gcT, xyz4, Wm1a, Wm1b, bm1, Wm2, bm2, Wpos, bpos, Wl, bl, Wq, Wk, Wv)


def _global_call(xn, q, k, v, Wg, bg):
    grid = (B_, N_ // TQ)
    tile = pl.BlockSpec((1, TQ, D_), lambda b, n: (b, n, 0))
    row = pl.BlockSpec((1, N_, D_), lambda b, n: (b, 0, 0))
    return pl.pallas_call(
        _global_body,
        grid=grid,
        in_specs=[tile, tile, row, row,
                  pl.BlockSpec((D_, D_), lambda b, n: (0, 0)),
                  pl.BlockSpec((1, D_), lambda b, n: (0, 0))],
        out_specs=tile,
        out_shape=jax.ShapeDtypeStruct((B_, N_, D_), F32),
    )(xn, q, k, v, Wg, bg)


def kernel(x, xyz, W_pos, b_pos, W_m1, b_m1, W_m2, b_m2,
           W_lproj, b_lproj, W_q, W_k, W_v, W_gproj, b_gproj):
    xyz4 = jnp.pad(xyz, ((0, 0), (0, 0), (0, 1)))
    n2 = jnp.sum(xyz * xyz, -1)[:, None, :]
    idxg = _topk_call(xyz4, n2)
    idxTg = jnp.swapaxes(idxg, 1, 2)
    xc = jnp.concatenate(
        [x, jnp.pad(xyz, ((0, 0), (0, 0), (0, _DC - D_ - 3)))], axis=-1)
    gc_flat = _sc_gather(xc.reshape(B_ * N_, _DC),
                         idxTg.reshape(_ROWS).astype(jnp.int32))
    gcT = gc_flat.reshape(B_, K_, N_, _DC)

    r2 = lambda a: a.reshape(1, -1)
    xn, q, k, v = _local_call(
        x, gcT, xyz4,
        W_m1[:D_], W_m1[D_:], r2(b_m1), W_m2, r2(b_m2),
        W_pos, r2(b_pos), W_lproj, r2(b_lproj), W_q, W_k, W_v)

    return _global_call(xn, q, k, v, W_gproj, r2(b_gproj))


# speedup vs baseline: 3.9764x; 1.1181x over previous
import functools

import jax
import jax.numpy as jnp
from jax import lax
from jax.experimental import pallas as pl
from jax.experimental.pallas import tpu as pltpu
from jax.experimental.pallas import tpu_sc as plsc

B_, N_, D_, H_, K_ = 4, 2048, 384, 8, 16
HD_ = D_ // H_
CBN = (1.0 + 1e-5) ** -0.5
SCALE = HD_ ** -0.5
TN = 256
TQ = 256
F32 = jnp.float32
_P = jax.lax.Precision.DEFAULT


def _relu(v):
    return jnp.maximum(v, 0.0)


def _dot(a, b, prec=_P):
    return jax.lax.dot_general(a, b, (((1,), (0,)), ((), ())),
                               precision=prec, preferred_element_type=F32)


def _dot_t(a, b, prec=_P):
    return jax.lax.dot_general(a, b, (((1,), (1,)), ((), ())),
                               precision=prec, preferred_element_type=F32)


_NW = 32
_ROWS = B_ * K_ * N_
_PER_W = _ROWS // _NW
_CH = 128
_NCH = _PER_W // _CH


_DC = 512


def _sc_gather_body(per_w, nch, xc_hbm, idx_hbm, gx_hbm, idx_v, rows_v, sem1):
    wid = lax.axis_index("s") * 2 + lax.axis_index("c")
    base = wid * per_w

    def chunk(j, carry):
        b = base + j * _CH
        pltpu.sync_copy(idx_hbm.at[pl.ds(b, _CH)], idx_v)
        pltpu.async_copy(xc_hbm.at[idx_v], rows_v, sem1).wait()
        pltpu.sync_copy(rows_v, gx_hbm.at[pl.ds(b, _CH)])
        return carry

    lax.fori_loop(0, nch, chunk, 0)


def _sc_gather(xc, idxTg):
    rows = idxTg.shape[0]
    per_w = rows // _NW
    nch = per_w // _CH
    mesh = plsc.VectorSubcoreMesh(core_axis_name="c", subcore_axis_name="s")
    f = pl.kernel(
        functools.partial(_sc_gather_body, per_w, nch),
        mesh=mesh,
        out_type=jax.ShapeDtypeStruct((rows, _DC), F32),
        scratch_types=[
            pltpu.VMEM((_CH,), jnp.int32),
            pltpu.VMEM((_CH, _DC), F32),
            pltpu.SemaphoreType.DMA,
        ],
    )
    return f(xc, idxTg)


TR = 256


def _topk_body(xyz4_ref, xyzall_ref, n2_ref, idx_ref):
    b = pl.program_id(0)
    xt = xyz4_ref[0]
    n2t = jnp.sum(xt * xt, axis=1, keepdims=True)
    d = n2t + n2_ref[0] - 2.0 * _dot_t(xt, xyzall_ref[0])
    lane_n = jax.lax.broadcasted_iota(jnp.int32, (TR, N_), 1)
    lane_k = jax.lax.broadcasted_iota(jnp.int32, (TR, K_), 1)
    idxs = jnp.zeros((TR, K_), jnp.int32)
    for kk in range(K_):
        m = jnp.min(d, axis=1, keepdims=True)
        cand = jnp.where(d == m, lane_n, N_)
        a = jnp.min(cand, axis=1, keepdims=True)
        idxs = jnp.where(lane_k == kk, a + b * N_, idxs)
        d = jnp.where(lane_n == a, float('inf'), d)
    idx_ref[0] = idxs


def _topk_call(xyz4, n2):
    nb = xyz4.shape[0]
    grid = (nb, N_ // TR)
    return pl.pallas_call(
        _topk_body,
        grid=grid,
        in_specs=[
            pl.BlockSpec((1, TR, 4), lambda b, n: (b, n, 0)),
            pl.BlockSpec((1, N_, 4), lambda b, n: (b, 0, 0)),
            pl.BlockSpec((1, 1, N_), lambda b, n: (b, 0, 0)),
        ],
        out_specs=pl.BlockSpec((1, TR, K_), lambda b, n: (b, n, 0)),
        out_shape=jax.ShapeDtypeStruct((nb, N_, K_), jnp.int32),
    )(xyz4, xyz4, n2)


def _local_body(x_ref, gc_ref, xyz4_ref,
                Wm1a_ref, Wm1b_ref, bm1_ref, Wm2_ref, bm2_ref,
                Wpos_ref, bpos_ref, Wl_ref, bl_ref,
                Wq_ref, Wk_ref, Wv_ref,
                xn_ref, q_ref, k_ref, v_ref):
    x = x_ref[0]
    gc = gc_ref[0]
    gx = gc[..., 0:D_].reshape(K_ * TN, D_)

    g4 = gc[..., D_:D_ + 4]
    rel = g4 - g4[0:1]
    d2 = jnp.sum(rel * rel, -1, keepdims=True)
    lane4 = jax.lax.broadcasted_iota(jnp.int32, (K_, TN, 4), 2)
    rel4 = jnp.where(lane4 == 3, d2, rel).reshape(K_ * TN, 4)

    pos = _dot(rel4, Wpos_ref[...]) + bpos_ref[...]
    keyf = gx + pos

    a1 = _dot(_relu(keyf * CBN).astype(jnp.bfloat16),
              Wm1b_ref[...].astype(jnp.bfloat16))
    tq = _dot(_relu(x * CBN), Wm1a_ref[...])
    h1 = (a1.reshape(K_, TN, D_) + tq[None] + bm1_ref[...]).reshape(K_ * TN, D_)
    logits = (_dot(_relu(h1 * CBN), Wm2_ref[...]) + bm2_ref[...]) * SCALE

    lane = jax.lax.broadcasted_iota(jnp.int32, (H_, D_), 1)
    hid = jax.lax.broadcasted_iota(jnp.int32, (H_, D_), 0)
    E = (lane // HD_ == hid).astype(F32)

    m = logits[0:TN]
    for kk in range(1, K_):
        m = jnp.maximum(m, logits[kk * TN:(kk + 1) * TN])
    s = jnp.zeros((TN, H_), F32)
    acc = jnp.zeros((TN, D_), F32)
    for kk in range(K_):
        p = jnp.exp(logits[kk * TN:(kk + 1) * TN] - m)
        s = s + p
        acc = acc + _dot(p, E) * keyf[kk * TN:(kk + 1) * TN]
    out = acc / _dot(s, E)

    o = _dot(_relu(out * CBN), Wl_ref[...]) + bl_ref[...]
    xn = x + o
    xn_ref[0] = xn
    q_ref[0] = _dot(xn, Wq_ref[...]) * SCALE
    k_ref[0] = _dot(xn, Wk_ref[...])
    v_ref[0] = _dot(xn, Wv_ref[...])


def _global_body(xn_ref, q_ref, k_ref, v_ref, Wg_ref, bg_ref, out_ref):
    q = q_ref[0]
    kf = k_ref[0]
    vf = v_ref[0]
    acc = jnp.zeros((TQ, D_), F32)
    for h in range(H_):
        sl = slice(h * HD_, (h + 1) * HD_)
        sc = _dot_t(q[:, sl], kf[:, sl])
        m = jnp.max(sc, axis=1, keepdims=True)
        p = jnp.exp(sc - m)
        den = jnp.sum(p, axis=1, keepdims=True)
        sv = _dot(p, vf[:, sl])
        acc = acc + _dot(sv / den, Wg_ref[sl, :])
    out_ref[0] = xn_ref[0] + acc + bg_ref[...]


def _local_call(x, gcT, xyz4, Wm1a, Wm1b, bm1, Wm2, bm2,
                Wpos, bpos, Wl, bl, Wq, Wk, Wv):
    nb = x.shape[0]
    grid = (nb, N_ // TN)
    full = lambda shape: pl.BlockSpec(shape, lambda b, n: (0,) * len(shape))
    out_bs = pl.BlockSpec((1, TN, D_), lambda b, n: (b, n, 0))
    return pl.pallas_call(
        _local_body,
        grid=grid,
        in_specs=[
            pl.BlockSpec((1, TN, D_), lambda b, n: (b, n, 0)),
            pl.BlockSpec((1, K_, TN, _DC), lambda b, n: (b, 0, n, 0)),
            pl.BlockSpec((1, TN, 4), lambda b, n: (b, n, 0)),
            full((D_, D_)), full((D_, D_)), full((1, D_)),
            full((D_, H_)), full((1, H_)),
            full((4, D_)), full((1, D_)),
            full((D_, D_)), full((1, D_)),
            full((D_, D_)), full((D_, D_)), full((D_, D_)),
        ],
        out_specs=[out_bs, out_bs, out_bs, out_bs],
        out_shape=[jax.ShapeDtypeStruct((nb, N_, D_), F32)] * 4,
    )(x, gcT, xyz4, Wm1a, Wm1b, bm1, Wm2, bm2, Wpos, bpos, Wl, bl, Wq, Wk, Wv)


def _global_call(xn, q, k, v, Wg, bg):
    nb = xn.shape[0]
    grid = (nb, N_ // TQ)
    tile = pl.BlockSpec((1, TQ, D_), lambda b, n: (b, n, 0))
    row = pl.BlockSpec((1, N_, D_), lambda b, n: (b, 0, 0))
    return pl.pallas_call(
        _global_body,
        grid=grid,
        in_specs=[tile, tile, row, row,
                  pl.BlockSpec((D_, D_), lambda b, n: (0, 0)),
                  pl.BlockSpec((1, D_), lambda b, n: (0, 0))],
        out_specs=tile,
        out_shape=jax.ShapeDtypeStruct((nb, N_, D_), F32),
    )(xn, q, k, v, Wg, bg)


def kernel(x, xyz, W_pos, b_pos, W_m1, b_m1, W_m2, b_m2,
           W_lproj, b_lproj, W_q, W_k, W_v, W_gproj, b_gproj):
    r2 = lambda a: a.reshape(1, -1)
    xyz4 = jnp.pad(xyz, ((0, 0), (0, 0), (0, 1)))
    n2 = jnp.sum(xyz * xyz, -1)[:, None, :]
    xc = jnp.concatenate(
        [x, jnp.pad(xyz, ((0, 0), (0, 0), (0, _DC - D_ - 3)))], axis=-1)

    outs = []
    for b in range(B_):
        xb = x[b:b + 1]
        idxg = _topk_call(xyz4[b:b + 1], n2[b:b + 1])
        idxTg = jnp.swapaxes(idxg, 1, 2)
        gc_flat = _sc_gather(xc[b], idxTg.reshape(K_ * N_).astype(jnp.int32))
        gcT = gc_flat.reshape(1, K_, N_, _DC)
        xn, q, k, v = _local_call(
            xb, gcT, xyz4[b:b + 1],
            W_m1[:D_], W_m1[D_:], r2(b_m1), W_m2, r2(b_m2),
            W_pos, r2(b_pos), W_lproj, r2(b_lproj), W_q, W_k, W_v)
        outs.append(_global_call(xn, q, k, v, W_gproj, r2(b_gproj)))
    return jnp.concatenate(outs, axis=0)


# speedup vs baseline: 4.1798x; 1.0512x over previous
import functools

import jax
import jax.numpy as jnp
from jax import lax
from jax.experimental import pallas as pl
from jax.experimental.pallas import tpu as pltpu
from jax.experimental.pallas import tpu_sc as plsc

B_, N_, D_, H_, K_ = 4, 2048, 384, 8, 16
HD_ = D_ // H_
CBN = (1.0 + 1e-5) ** -0.5
SCALE = HD_ ** -0.5
TN = 256
TQ = 512
F32 = jnp.float32
_P = jax.lax.Precision.DEFAULT


def _relu(v):
    return jnp.maximum(v, 0.0)


def _dot(a, b, prec=_P):
    return jax.lax.dot_general(a, b, (((1,), (0,)), ((), ())),
                               precision=prec, preferred_element_type=F32)


def _dot_t(a, b, prec=_P):
    return jax.lax.dot_general(a, b, (((1,), (1,)), ((), ())),
                               precision=prec, preferred_element_type=F32)


_NW = 32
_ROWS = B_ * K_ * N_
_PER_W = _ROWS // _NW
_CH = 128
_NCH = _PER_W // _CH


_DC = 512


def _sc_gather_body(per_w, nch, xc_hbm, idx_hbm, gx_hbm, idx_v, rows_v, sem1):
    wid = lax.axis_index("s") * 2 + lax.axis_index("c")
    base = wid * per_w

    def chunk(j, carry):
        b = base + j * _CH
        pltpu.sync_copy(idx_hbm.at[pl.ds(b, _CH)], idx_v)
        pltpu.async_copy(xc_hbm.at[idx_v], rows_v, sem1).wait()
        pltpu.sync_copy(rows_v, gx_hbm.at[pl.ds(b, _CH)])
        return carry

    lax.fori_loop(0, nch, chunk, 0)


def _sc_gather(xc, idxTg):
    rows = idxTg.shape[0]
    per_w = rows // _NW
    nch = per_w // _CH
    mesh = plsc.VectorSubcoreMesh(core_axis_name="c", subcore_axis_name="s")
    f = pl.kernel(
        functools.partial(_sc_gather_body, per_w, nch),
        mesh=mesh,
        out_type=jax.ShapeDtypeStruct((rows, _DC), F32),
        scratch_types=[
            pltpu.VMEM((_CH,), jnp.int32),
            pltpu.VMEM((_CH, _DC), F32),
            pltpu.SemaphoreType.DMA,
        ],
    )
    return f(xc, idxTg)


TR = 512


def _topk_body(xyz4_ref, xyzall_ref, n2_ref, idx_ref):
    b = pl.program_id(0)
    xt = xyz4_ref[0]
    n2t = jnp.sum(xt * xt, axis=1, keepdims=True)
    d = n2t + n2_ref[0] - 2.0 * _dot_t(xt, xyzall_ref[0])
    lane_n = jax.lax.broadcasted_iota(jnp.int32, (TR, N_), 1)
    lane_k = jax.lax.broadcasted_iota(jnp.int32, (TR, K_), 1)
    idxs = jnp.zeros((TR, K_), jnp.int32)
    for kk in range(K_):
        m = jnp.min(d, axis=1, keepdims=True)
        cand = jnp.where(d == m, lane_n, N_)
        a = jnp.min(cand, axis=1, keepdims=True)
        idxs = jnp.where(lane_k == kk, a + b * N_, idxs)
        d = jnp.where(lane_n == a, float('inf'), d)
    idx_ref[0] = idxs


def _topk_call(xyz4, n2):
    nb = xyz4.shape[0]
    grid = (nb, N_ // TR)
    return pl.pallas_call(
        _topk_body,
        grid=grid,
        in_specs=[
            pl.BlockSpec((1, TR, 4), lambda b, n: (b, n, 0)),
            pl.BlockSpec((1, N_, 4), lambda b, n: (b, 0, 0)),
            pl.BlockSpec((1, 1, N_), lambda b, n: (b, 0, 0)),
        ],
        out_specs=pl.BlockSpec((1, TR, K_), lambda b, n: (b, n, 0)),
        out_shape=jax.ShapeDtypeStruct((nb, N_, K_), jnp.int32),
    )(xyz4, xyz4, n2)


def _local_body(x_ref, gc_ref, xyz4_ref,
                Wm1a_ref, Wm1b_ref, bm1_ref, Wm2_ref, bm2_ref,
                Wpos_ref, bpos_ref, Wl_ref, bl_ref,
                Wq_ref, Wk_ref, Wv_ref,
                xn_ref, q_ref, k_ref, v_ref):
    x = x_ref[0]
    gc = gc_ref[0]
    gx = gc[..., 0:D_].reshape(K_ * TN, D_)

    g4 = gc[..., D_:D_ + 4]
    rel = g4 - g4[0:1]
    d2 = jnp.sum(rel * rel, -1, keepdims=True)
    lane4 = jax.lax.broadcasted_iota(jnp.int32, (K_, TN, 4), 2)
    rel4 = jnp.where(lane4 == 3, d2, rel).reshape(K_ * TN, 4)

    pos = _dot(rel4, Wpos_ref[...]) + bpos_ref[...]
    keyf = gx + pos

    a1 = _dot(_relu(keyf * CBN).astype(jnp.bfloat16),
              Wm1b_ref[...].astype(jnp.bfloat16))
    tq = _dot(_relu(x * CBN), Wm1a_ref[...])
    h1 = (a1.reshape(K_, TN, D_) + tq[None] + bm1_ref[...]).reshape(K_ * TN, D_)
    logits = (_dot(_relu(h1 * CBN), Wm2_ref[...]) + bm2_ref[...]) * SCALE

    lane = jax.lax.broadcasted_iota(jnp.int32, (H_, D_), 1)
    hid = jax.lax.broadcasted_iota(jnp.int32, (H_, D_), 0)
    E = (lane // HD_ == hid).astype(F32)

    m = logits[0:TN]
    for kk in range(1, K_):
        m = jnp.maximum(m, logits[kk * TN:(kk + 1) * TN])
    s = jnp.zeros((TN, H_), F32)
    acc = jnp.zeros((TN, D_), F32)
    for kk in range(K_):
        p = jnp.exp(logits[kk * TN:(kk + 1) * TN] - m)
        s = s + p
        acc = acc + _dot(p, E) * keyf[kk * TN:(kk + 1) * TN]
    out = acc / _dot(s, E)

    o = _dot(_relu(out * CBN), Wl_ref[...]) + bl_ref[...]
    xn = x + o
    xn_ref[0] = xn
    q_ref[0] = _dot(xn, Wq_ref[...]) * SCALE
    k_ref[0] = _dot(xn, Wk_ref[...])
    v_ref[0] = _dot(xn, Wv_ref[...])


def _global_body(xn_ref, q_ref, k_ref, v_ref, Wg_ref, bg_ref, out_ref):
    q = q_ref[0]
    kf = k_ref[0]
    vf = v_ref[0]
    acc = jnp.zeros((TQ, D_), F32)
    for h in range(H_):
        sl = slice(h * HD_, (h + 1) * HD_)
        sc = _dot_t(q[:, sl], kf[:, sl])
        m = jnp.max(sc, axis=1, keepdims=True)
        p = jnp.exp(sc - m)
        den = jnp.sum(p, axis=1, keepdims=True)
        sv = _dot(p, vf[:, sl])
        acc = acc + _dot(sv / den, Wg_ref[sl, :])
    out_ref[0] = xn_ref[0] + acc + bg_ref[...]


def _local_call(x, gcT, xyz4, Wm1a, Wm1b, bm1, Wm2, bm2,
                Wpos, bpos, Wl, bl, Wq, Wk, Wv):
    nb = x.shape[0]
    grid = (nb, N_ // TN)
    full = lambda shape: pl.BlockSpec(shape, lambda b, n: (0,) * len(shape))
    out_bs = pl.BlockSpec((1, TN, D_), lambda b, n: (b, n, 0))
    return pl.pallas_call(
        _local_body,
        grid=grid,
        in_specs=[
            pl.BlockSpec((1, TN, D_), lambda b, n: (b, n, 0)),
            pl.BlockSpec((1, K_, TN, _DC), lambda b, n: (b, 0, n, 0)),
            pl.BlockSpec((1, TN, 4), lambda b, n: (b, n, 0)),
            full((D_, D_)), full((D_, D_)), full((1, D_)),
            full((D_, H_)), full((1, H_)),
            full((4, D_)), full((1, D_)),
            full((D_, D_)), full((1, D_)),
            full((D_, D_)), full((D_, D_)), full((D_, D_)),
        ],
        out_specs=[out_bs, out_bs, out_bs, out_bs],
        out_shape=[jax.ShapeDtypeStruct((nb, N_, D_), F32)] * 4,
    )(x, gcT, xyz4, Wm1a, Wm1b, bm1, Wm2, bm2, Wpos, bpos, Wl, bl, Wq, Wk, Wv)


def _global_call(xn, q, k, v, Wg, bg):
    nb = xn.shape[0]
    grid = (nb, N_ // TQ)
    tile = pl.BlockSpec((1, TQ, D_), lambda b, n: (b, n, 0))
    row = pl.BlockSpec((1, N_, D_), lambda b, n: (b, 0, 0))
    return pl.pallas_call(
        _global_body,
        grid=grid,
        in_specs=[tile, tile, row, row,
                  pl.BlockSpec((D_, D_), lambda b, n: (0, 0)),
                  pl.BlockSpec((1, D_), lambda b, n: (0, 0))],
        out_specs=tile,
        out_shape=jax.ShapeDtypeStruct((nb, N_, D_), F32),
    )(xn, q, k, v, Wg, bg)


def kernel(x, xyz, W_pos, b_pos, W_m1, b_m1, W_m2, b_m2,
           W_lproj, b_lproj, W_q, W_k, W_v, W_gproj, b_gproj):
    r2 = lambda a: a.reshape(1, -1)
    xyz4 = jnp.pad(xyz, ((0, 0), (0, 0), (0, 1)))
    n2 = jnp.sum(xyz * xyz, -1)[:, None, :]
    xc = jnp.concatenate(
        [x, jnp.pad(xyz, ((0, 0), (0, 0), (0, _DC - D_ - 3)))], axis=-1)

    outs = []
    for b in range(B_):
        xb = x[b:b + 1]
        idxg = _topk_call(xyz4[b:b + 1], n2[b:b + 1])
        idxTg = jnp.swapaxes(idxg, 1, 2)
        gc_flat = _sc_gather(xc[b], idxTg.reshape(K_ * N_).astype(jnp.int32))
        gcT = gc_flat.reshape(1, K_, N_, _DC)
        xn, q, k, v = _local_call(
            xb, gcT, xyz4[b:b + 1],
            W_m1[:D_], W_m1[D_:], r2(b_m1), W_m2, r2(b_m2),
            W_pos, r2(b_pos), W_lproj, r2(b_lproj), W_q, W_k, W_v)
        outs.append(_global_call(xn, q, k, v, W_gproj, r2(b_gproj)))
    return jnp.concatenate(outs, axis=0)
